# baseline (speedup 1.0000x reference)
import functools

import jax
import jax.numpy as jnp
from jax import lax
from jax.experimental import pallas as pl
from jax.experimental.pallas import tpu as pltpu
from jax.experimental.pallas import tpu_sc as plsc

NC, NS = 2, 16
NW = NC * NS
NB = 2


@functools.partial(jax.jit, static_argnums=(2, 3, 4, 5))
def _gather(idx, table, b, t, v, d):
    rows_per_w = b // NW
    steps = rows_per_w // NB
    assert steps % 2 == 0 and steps >= 4
    pieces = [(o, min(128, t - o)) for o in range(0, t, 128)]
    mesh = plsc.VectorSubcoreMesh(
        core_axis_name="c", subcore_axis_name="s", num_cores=NC, num_subcores=NS
    )

    @functools.partial(
        pl.kernel,
        out_type=jax.ShapeDtypeStruct((b, t, d), jnp.float32),
        mesh=mesh,
        scratch_types=[
            pltpu.VMEM_SHARED((v, d), jnp.float32),
            pltpu.VMEM((NB, t), jnp.int32),
            pltpu.VMEM((NB, t), jnp.int32),
            pltpu.VMEM((NB, t, d), jnp.float32),
            pltpu.VMEM((NB, t, d), jnp.float32),
            pltpu.SemaphoreType.DMA,
            pltpu.SemaphoreType.DMA,
            pltpu.SemaphoreType.DMA,
            pltpu.SemaphoreType.DMA,
            pltpu.SemaphoreType.DMA,
            pltpu.SemaphoreType.DMA,
        ],
        compiler_params=pltpu.CompilerParams(use_tc_tiling_on_sc=False),
    )
    def k(idx_hbm, table_hbm, out_hbm,
          table_sh, idx0, idx1, rows0, rows1, si0, si1, sg0, sg1, so0, so1):
        sid = lax.axis_index("s")
        wid = sid * NC + lax.axis_index("c")
        base_row = wid * rows_per_w
        bufs = ((idx0, rows0, si0, sg0, so0), (idx1, rows1, si1, sg1, so1))

        @pl.when(sid == 0)
        def _():
            pltpu.sync_copy(table_hbm, table_sh)

        plsc.subcore_barrier()

        def issue_idx(g, bf):
            idx_v, _, si, _, _ = bufs[bf]
            pltpu.async_copy(idx_hbm.at[pl.ds(base_row + g * NB, NB)], idx_v, si)

        def run_chunk(g, bf, wait_out, next_idx):
            idx_v, rows_v, si, sg, so = bufs[bf]
            pltpu.make_async_copy(idx_hbm.at[pl.ds(0, NB)], idx_v, si).wait()
            if wait_out:
                pltpu.make_async_copy(
                    rows_v, out_hbm.at[pl.ds(0, NB)], so).wait()
            copies = [
                pltpu.async_copy(
                    table_sh.at[idx_v.at[r, pl.ds(o, n)]],
                    rows_v.at[r, pl.ds(o, n)],
                    sg,
                )
                for r in range(NB)
                for (o, n) in pieces
            ]
            for c in copies:
                c.wait()
            if next_idx:
                issue_idx(g + 2, bf)
            pltpu.async_copy(
                rows_v, out_hbm.at[pl.ds(base_row + g * NB, NB)], so)

        issue_idx(0, 0)
        issue_idx(1, 1)
        run_chunk(0, 0, wait_out=False, next_idx=True)
        run_chunk(1, 1, wait_out=False, next_idx=True)

        def outer(o, carry):
            g = o * 2
            run_chunk(g, 0, wait_out=True, next_idx=True)
            run_chunk(g + 1, 1, wait_out=True, next_idx=True)
            return carry

        lax.fori_loop(1, steps // 2 - 1, outer, 0)

        run_chunk(steps - 2, 0, wait_out=True, next_idx=False)
        run_chunk(steps - 1, 1, wait_out=True, next_idx=False)
        for bf in (0, 1):
            _, rows_v, _, _, so = bufs[bf]
            pltpu.make_async_copy(rows_v, out_hbm.at[pl.ds(0, NB)], so).wait()

    return k(idx, table)


def kernel(segment_ids, weight):
    b, t = segment_ids.shape
    v, d = weight.shape
    return _gather(segment_ids.astype(jnp.int32), weight, b, t, v, d)
